# baseline (speedup 1.0000x reference)
import functools
import math

import jax
import jax.numpy as jnp
from jax.experimental import pallas as pl
from jax.experimental.pallas import tpu as pltpu

_GAMMA = 0.1
_BIG = 1e6
_C1 = -math.log2(math.e) / _GAMMA
_C2 = -_GAMMA * math.log(2.0)
_BIGS = _BIG * _C1


def _sdtw_kernel(x_ref, y_ref, out_ref, *, N, M):
    C = N // 2

    def softmin_update(d, b0, b1, b2):
        bmax = jnp.maximum(jnp.maximum(b0, b1), b2)
        rsum = jnp.exp2(b0 - bmax) + jnp.exp2(b1 - bmax) + jnp.exp2(b2 - bmax)
        return (jnp.log2(rsum) + bmax) - d

    def make_bodies(U0, L):
        bigs_row = x_ref[:1, :] * 0.0 + _BIGS

        def dist(xoff, t):
            xw = x_ref[pl.ds(U0 + t + xoff, L), :]
            yw = y_ref[pl.ds(U0 + N - t, L), :]
            return jnp.abs(xw - yw)

        def body_even(t, wm2, wm1):
            d = dist(-1, t)
            b1 = jnp.concatenate([bigs_row, wm1[:-1, :]], axis=0)
            return softmin_update(d, wm2, b1, wm1)

        def body_odd(t, wm2, wm1):
            d = dist(0, t)
            b2 = jnp.concatenate([wm1[1:, :], bigs_row], axis=0)
            return softmin_update(d, wm2, wm1, b2)

        def pair(t, carry):
            a, b = carry
            c = body_even(t, a, b)
            return (c, body_odd(t, b, c))

        return body_even, pair

    def shrink_step(t, A, B, lo, L, U0):
        xw = x_ref[pl.ds(U0 + t, L), :]
        yw = y_ref[pl.ds(U0 + N - t, L), :]
        d = jnp.abs(xw - yw)
        return softmin_update(d, A[lo:lo + L, :], B[lo:lo + L, :],
                              B[lo + 1:lo + L + 1, :])

    def grow(w, add):
        pad = w[:add, :] * 0.0 + _BIGS
        return jnp.concatenate([pad, w, pad], axis=0)

    E = N // 8

    rows = jax.lax.broadcasted_iota(jnp.int32, (E, x_ref.shape[1]), 0)
    bigs_e = x_ref[pl.ds(0, E), :] * 0.0 + _BIGS
    w0 = jnp.where(rows == E // 2, 0.0, bigs_e)
    w1 = bigs_e

    be_e, pair_e = make_bodies(C - E // 2, E)
    a, b = jax.lax.fori_loop(1, 33, pair_e, (w0, w1), unroll=4)

    be_q, pair_q = make_bodies(C - E, 2 * E)
    a, b = jax.lax.fori_loop(33, 65, pair_q, (grow(a, E // 2), grow(b, E // 2)),
                             unroll=4)

    be_h, pair_h = make_bodies(C - 2 * E, 4 * E)
    a, b = jax.lax.fori_loop(65, 129, pair_h, (grow(a, E), grow(b, E)),
                             unroll=4)

    be_f, pair_f = make_bodies(0, N)
    a, b = jax.lax.fori_loop(129, 384, pair_f,
                             (grow(a, 2 * E), grow(b, 2 * E)), unroll=4)
    w768 = be_f(384, a, b)

    lo = C - 2 * E
    w769 = shrink_step(384, b, w768, lo, 4 * E, lo)
    a, b = jax.lax.fori_loop(385, 448, pair_h,
                             (w768[lo:lo + 4 * E, :], w769), unroll=4)
    w896 = be_h(448, a, b)

    w897 = shrink_step(448, b, w896, E, 2 * E, C - E)
    a, b = jax.lax.fori_loop(449, 480, pair_q, (w896[E:3 * E, :], w897),
                             unroll=4)
    w960 = be_q(480, a, b)

    w961 = shrink_step(480, b, w960, E // 2, E, C - E // 2)
    a, b = jax.lax.fori_loop(481, 512, pair_e, (w960[E // 2:3 * E // 2, :], w961),
                             unroll=4)
    w_last = be_e(512, a, b)

    out_ref[0, 0, :] = w_last[E // 2, :] * _C2


def kernel(x, y):
    B, N = x.shape
    M = y.shape[1]
    scale = jnp.float32(-_C1)
    xs = (x * scale).T
    ys = (y * scale)[:, ::-1].T
    C = N // 2
    x_pad = jnp.zeros((2 * N, B), jnp.float32).at[C:C + N].set(xs)
    y_pad = jnp.zeros((N + M, B), jnp.float32).at[C:C + M].set(ys)

    Bb = 128
    NB = B // Bb
    out = pl.pallas_call(
        functools.partial(_sdtw_kernel, N=N, M=M),
        grid=(NB,),
        in_specs=[
            pl.BlockSpec((2 * N, Bb), lambda i: (0, i)),
            pl.BlockSpec((N + M, Bb), lambda i: (0, i)),
        ],
        out_specs=pl.BlockSpec((1, 1, Bb), lambda i: (i, 0, 0)),
        out_shape=jax.ShapeDtypeStruct((NB, 1, Bb), jnp.float32),
        compiler_params=pltpu.CompilerParams(dimension_semantics=("parallel",)),
    )(x_pad, y_pad)
    loss = out.reshape(B) / (N + M)
    return loss.mean()


# speedup vs baseline: 1.0874x; 1.0874x over previous
import functools
import math

import jax
import jax.numpy as jnp
from jax.experimental import pallas as pl
from jax.experimental.pallas import tpu as pltpu

_GAMMA = 0.1
_BIG = 1e6
_C1 = -math.log2(math.e) / _GAMMA
_C2 = -_GAMMA * math.log(2.0)


def _sdtw_kernel(x_ref, y_ref, out_ref, *, N, M):
    x = x_ref[:, :]
    big = x * 0.0 + _BIG * _C1
    big_row = big[:1, :]
    zero_row = big_row * 0.0
    H = N // 2

    def make_body(xs, off):
        L = xs.shape[0]

        def body(k, v_km2, v_km1, r0_row, r1_row):
            yw = y_ref[pl.ds(off + N + M - k, L), :]
            d = jnp.abs(xs - yw)
            b0 = jnp.concatenate([r0_row, v_km2[:-1, :]], axis=0)
            b1 = jnp.concatenate([r1_row, v_km1[:-1, :]], axis=0)
            b2 = v_km1
            bmax = jnp.maximum(jnp.maximum(b0, b1), b2)
            rsum = jnp.exp2(b0 - bmax) + jnp.exp2(b1 - bmax) + jnp.exp2(b2 - bmax)
            return (jnp.log2(rsum) + bmax) - d

        return body

    Q = N // 4

    def run(body, k_lo, k_hi, a, b, unroll=16):
        def step(k, carry):
            a, b = carry
            return (b, body(k, a, b, big_row, big_row))

        return jax.lax.fori_loop(k_lo, k_hi, step, (a, b), unroll=unroll)

    def shrink(body, k_first, a, b, cut):
        row_a = a[cut - 1:cut, :]
        row_b = b[cut - 1:cut, :]
        v0 = body(k_first, a[cut:, :], b[cut:, :], row_a, row_b)
        v1 = body(k_first + 1, b[cut:, :], v0, row_b, big_row)
        return v0, v1

    E = N // 8

    body_e0 = make_body(x[:E, :], 0)
    big_e = big[:E, :]
    v2 = body_e0(2, big_e, big_e, zero_row, big_row)
    a, b = run(body_e0, 3, E + 2, big_e, v2)

    body_q0 = make_body(x[:Q, :], 0)
    a, b = run(body_q0, E + 2, Q + 2,
               jnp.concatenate([a, big_e], axis=0),
               jnp.concatenate([b, big_e], axis=0))

    body_h0 = make_body(x[:H, :], 0)
    big_q = big[:Q, :]
    a, b = run(body_h0, Q + 2, H + 2,
               jnp.concatenate([a, big_q], axis=0),
               jnp.concatenate([b, big_q], axis=0))

    body_full = make_body(x, 0)
    big_h = big[:H, :]
    a, b = run(body_full, H + 2, N + H + 1,
               jnp.concatenate([a, big_h], axis=0),
               jnp.concatenate([b, big_h], axis=0))

    body_hi = make_body(x[H:, :], H)
    v0, v1 = shrink(body_hi, N + H + 1, a, b, H)
    a, b = run(body_hi, N + H + 3, N + M - Q + 1, v0, v1)

    body_q1 = make_body(x[N - Q:, :], N - Q)
    v0, v1 = shrink(body_q1, N + M - Q + 1, a, b, Q)
    a, b = run(body_q1, N + M - Q + 3, N + M - E + 1, v0, v1)

    body_e1 = make_body(x[N - E:, :], N - E)
    v0, v1 = shrink(body_e1, N + M - E + 1, a, b, Q - E)
    _, v_last = run(body_e1, N + M - E + 3, N + M + 1, v0, v1)
    out_ref[0, 0, :] = v_last[E - 1, :] * _C2


def kernel(x, y):
    B, N = x.shape
    M = y.shape[1]
    scale = jnp.float32(-_C1)
    x_t = (x * scale).T
    y_rev = (y * scale)[:, ::-1].T
    pad_left = N - 1
    total = pad_left + M + (N - 1)
    padded = ((total + 7) // 8) * 8
    y_pad = jnp.zeros((padded, B), jnp.float32).at[pad_left:pad_left + M].set(y_rev)

    Bb = 128
    NB = B // Bb
    out = pl.pallas_call(
        functools.partial(_sdtw_kernel, N=N, M=M),
        grid=(NB,),
        in_specs=[
            pl.BlockSpec((N, Bb), lambda i: (0, i)),
            pl.BlockSpec((padded, Bb), lambda i: (0, i)),
        ],
        out_specs=pl.BlockSpec((1, 1, Bb), lambda i: (i, 0, 0)),
        out_shape=jax.ShapeDtypeStruct((NB, 1, Bb), jnp.float32),
        compiler_params=pltpu.CompilerParams(dimension_semantics=("parallel",)),
    )(x_t, y_pad)
    loss = out.reshape(B) / (N + M)
    return loss.mean()
